# baseline (speedup 1.0000x reference)
import functools

import jax
import jax.numpy as jnp
from jax import lax
from jax.experimental import pallas as pl
from jax.experimental.pallas import tpu as pltpu
from jax.experimental.pallas import tpu_sc as plsc

N = 10000
E = 320000
D = 128
H = 128
G = 64
NC_OUT = 10
NB = 3

NCORES = 2
NSUB = 16
NW = NCORES * NSUB
K = 3
CHUNK = K * 128
NSTEPS = 27
EPW = NSTEPS * CHUNK
EPAD = EPW * NW
ACC_ROWS = 10112
ZROWS = ACC_ROWS // NSUB


@functools.partial(
    pl.kernel,
    out_type=jax.ShapeDtypeStruct((NB, NCORES, ACC_ROWS, D), jnp.float32),
    mesh=plsc.VectorSubcoreMesh(core_axis_name="c", subcore_axis_name="s"),
    scratch_types=[
        pltpu.VMEM((CHUNK,), jnp.int32),
        pltpu.VMEM((CHUNK,), jnp.int32),
        pltpu.VMEM((CHUNK, D), jnp.float32),
        pltpu.VMEM_SHARED((ACC_ROWS, D), jnp.float32),
        pltpu.SemaphoreType.DMA,
    ],
)
def _sc_agg3(x_hbm, src_hbm, dst_hbm, zeros_hbm, out_hbm,
             srcb, dstb, rows, acc, gsem):
    c = lax.axis_index("c")
    s = lax.axis_index("s")
    w = c * NSUB + s

    for br in range(NB):
        pltpu.sync_copy(zeros_hbm, acc.at[pl.ds(s * ZROWS, ZROWS)])
        plsc.subcore_barrier()

        def step(j, _):
            pltpu.sync_copy(src_hbm.at[br, w, j], srcb)
            pltpu.sync_copy(dst_hbm.at[br, w, j], dstb)
            pltpu.async_copy(x_hbm.at[br].at[srcb], rows, gsem).wait()
            pltpu.sync_copy(rows, acc.at[dstb], add=True)
            return 0

        lax.fori_loop(0, NSTEPS, step, 0)
        plsc.subcore_barrier()

        for off, sz in ((0, 312), (312, 320)):
            r0 = s * ZROWS + off
            buf = rows.at[pl.ds(0, sz)]
            pltpu.sync_copy(acc.at[pl.ds(r0, sz)], buf)
            pltpu.sync_copy(buf, out_hbm.at[br, c, pl.ds(r0, sz)])


BN = 2000
NBLK = N // BN


def _mm(a, b):
    return jnp.dot(a, b, preferred_element_type=jnp.float32,
                   precision=lax.Precision.HIGHEST)


def _gin_math(x, p0, p1, w1_ref, b1_ref, w2_ref, b2_ref):
    h = x + p0 + p1
    h = _mm(h, w1_ref[...]) + b1_ref[...]
    h = jnp.maximum(h, 0.0)
    h = _mm(h, w2_ref[...]) + b2_ref[...]
    return jnp.maximum(h, 0.0)


def _tc_layer_body(x_ref, p0_ref, p1_ref, w1_ref, b1_ref, w2_ref, b2_ref,
                   o_ref):
    o_ref[0] = _gin_math(x_ref[0], p0_ref[0, 0], p1_ref[0, 0],
                         w1_ref, b1_ref, w2_ref, b2_ref)


_X_SPECS = [
    pl.BlockSpec((1, BN, D), lambda b, i: (b, i, 0)),
    pl.BlockSpec((1, 1, BN, D), lambda b, i: (b, 0, i, 0)),
    pl.BlockSpec((1, 1, BN, D), lambda b, i: (b, 1, i, 0)),
]
_W_SPECS = [
    pl.BlockSpec((D, H), lambda b, i: (0, 0)),
    pl.BlockSpec((1, H), lambda b, i: (0, 0)),
    pl.BlockSpec((H, H), lambda b, i: (0, 0)),
    pl.BlockSpec((1, H), lambda b, i: (0, 0)),
]


def _tc_layer(x3, parts, w1, b1, w2, b2):
    return pl.pallas_call(
        _tc_layer_body,
        grid=(NB, NBLK),
        in_specs=_X_SPECS + _W_SPECS,
        out_specs=pl.BlockSpec((1, BN, H), lambda b, i: (b, i, 0)),
        out_shape=jax.ShapeDtypeStruct((NB, N, H), jnp.float32),
    )(x3, parts, parts, w1, b1, w2, b2)


def _tc_layer_pool_body(x_ref, p0_ref, p1_ref, w1_ref, b1_ref, w2_ref,
                        b2_ref, batch_ref, s_ref, c_ref):
    h = _gin_math(x_ref[0], p0_ref[0, 0], p1_ref[0, 0],
                  w1_ref, b1_ref, w2_ref, b2_ref)

    gids = lax.broadcasted_iota(jnp.int32, (BN, G), 1)
    onehot_t = (gids == batch_ref[0]).astype(jnp.float32)

    @pl.when(pl.program_id(1) == 0)
    def _():
        s_ref[...] = jnp.zeros_like(s_ref)
        c_ref[...] = jnp.zeros_like(c_ref)

    s_ref[0] += lax.dot_general(
        onehot_t, h, (((0,), (0,)), ((), ())),
        preferred_element_type=jnp.float32,
        precision=lax.Precision.HIGHEST)
    c_ref[0] += jnp.sum(onehot_t, axis=0)[None, :]


def _tc_layer_pool(x3, parts, w1, b1, w2, b2, batch3):
    return pl.pallas_call(
        _tc_layer_pool_body,
        grid=(NB, NBLK),
        in_specs=_X_SPECS + _W_SPECS + [
            pl.BlockSpec((1, BN, 1), lambda b, i: (b, i, 0))],
        out_specs=[
            pl.BlockSpec((1, G, H), lambda b, i: (b, 0, 0)),
            pl.BlockSpec((1, 1, G), lambda b, i: (b, 0, 0)),
        ],
        out_shape=[
            jax.ShapeDtypeStruct((NB, G, H), jnp.float32),
            jax.ShapeDtypeStruct((NB, 1, G), jnp.float32),
        ],
    )(x3, parts, parts, w1, b1, w2, b2, batch3)


def _tc_head_body(s3_ref, c3_ref, mw_ref, mb_ref, fw1_ref, fb1_ref,
                  fw2_ref, fb2_ref, o_ref):
    def hg(br):
        cnt = jnp.maximum(c3_ref[br], 1.0)
        pooled = s3_ref[br] / cnt.reshape(G, 1)
        return _mm(pooled, mw_ref[...]) + mb_ref[...]

    acc = (_mm(hg(0), fw1_ref[0:H, :])
           + _mm(hg(1), fw1_ref[H:2 * H, :])
           + _mm(hg(2), fw1_ref[2 * H:3 * H, :]))
    acc = jnp.maximum(acc + fb1_ref[...], 0.0)
    o_ref[...] = _mm(acc, fw2_ref[...]) + fb2_ref[...]


def _tc_head(s3, c3, mlp_W, mlp_b2, fW1, fb1_2, fW2, fb2_2):
    return pl.pallas_call(
        _tc_head_body,
        out_shape=jax.ShapeDtypeStruct((G, NC_OUT), jnp.float32),
    )(s3, c3, mlp_W, mlp_b2, fW1, fb1_2, fW2, fb2_2)


def kernel(x_org, edge_index_org, batch_org, x_c1, edge_index_c1, batch_c1,
           x_c2, edge_index_c2, batch_c2, conv_W1, conv_b1, conv_W2, conv_b2,
           mlp_W, mlp_b, final_W1, final_b1, final_W2, final_b2):
    zeros = jnp.zeros((ZROWS, D), jnp.float32)
    pad_src = jnp.zeros((EPAD - E,), jnp.int32)
    pad_dst = N + (jnp.arange(EPAD - E, dtype=jnp.int32) % (ACC_ROWS - N))

    def prep(ei):
        src = jnp.concatenate([ei[0], pad_src]).reshape(NW, NSTEPS, CHUNK)
        dst = jnp.concatenate([ei[1], pad_dst]).reshape(NW, NSTEPS, CHUNK)
        return src, dst

    s_o, d_o = prep(edge_index_org)
    s_1, d_1 = prep(edge_index_c1)
    s_2, d_2 = prep(edge_index_c2)
    src3 = jnp.stack([s_o, s_1, s_2])
    dst3 = jnp.stack([d_o, d_1, d_2])
    x3 = jnp.stack([x_org, x_c1, x_c2])
    batch3 = jnp.stack([batch_org, batch_c1, batch_c2]).reshape(NB, N, 1)

    b1r = conv_b1.reshape(3, 1, H)
    b2r = conv_b2.reshape(3, 1, H)

    for l in range(2):
        parts = _sc_agg3(x3, src3, dst3, zeros)
        x3 = _tc_layer(x3, parts, conv_W1[l], b1r[l], conv_W2[l], b2r[l])
    parts = _sc_agg3(x3, src3, dst3, zeros)
    s3, c3 = _tc_layer_pool(x3, parts, conv_W1[2], b1r[2], conv_W2[2],
                            b2r[2], batch3)

    return _tc_head(s3, c3,
                    mlp_W, mlp_b.reshape(1, H),
                    final_W1, final_b1.reshape(1, H),
                    final_W2, final_b2.reshape(1, NC_OUT))


# speedup vs baseline: 1.3093x; 1.3093x over previous
import functools

import jax
import jax.numpy as jnp
from jax import lax
from jax.experimental import pallas as pl
from jax.experimental.pallas import tpu as pltpu
from jax.experimental.pallas import tpu_sc as plsc

N = 10000
E = 320000
D = 128
H = 128
G = 64
NC_OUT = 10
NB = 3

NCORES = 2
NSUB = 16
NW = NCORES * NSUB
CHUNK = 128
NSTEPS = 80
EPW = NSTEPS * CHUNK
EPAD = EPW * NW
NIC = NSTEPS + 1
ACC_ROWS = 10112
ZROWS = ACC_ROWS // NSUB


@functools.partial(
    pl.kernel,
    out_type=jax.ShapeDtypeStruct((NB, NCORES, ACC_ROWS, D), jnp.float32),
    mesh=plsc.VectorSubcoreMesh(core_axis_name="c", subcore_axis_name="s"),
    scratch_types=[
        [pltpu.VMEM((CHUNK,), jnp.int32)] * 2,
        [pltpu.VMEM((CHUNK,), jnp.int32)] * 2,
        [pltpu.VMEM((CHUNK, D), jnp.float32)] * 2,
        pltpu.VMEM_SHARED((ACC_ROWS, D), jnp.float32),
        pltpu.SemaphoreType.DMA,
        [pltpu.SemaphoreType.DMA] * 2,
    ],
)
def _sc_agg3(x_hbm, src_hbm, dst_hbm, zeros_hbm, out_hbm,
             srcb, dstb, rows, acc, gsem, ssems):
    c = lax.axis_index("c")
    s = lax.axis_index("s")
    w = c * NSUB + s

    for br in range(NB):
        pltpu.sync_copy(zeros_hbm, acc.at[pl.ds(s * ZROWS, ZROWS)])
        pltpu.sync_copy(src_hbm.at[br, w, 0], srcb[0])
        pltpu.sync_copy(dst_hbm.at[br, w, 0], dstb[0])
        plsc.subcore_barrier()

        def pair(g, _):
            dg0 = pltpu.async_copy(x_hbm.at[br].at[srcb[0]], rows[0], gsem)
            pltpu.sync_copy(src_hbm.at[br, w, 2 * g + 1], srcb[1])
            pltpu.sync_copy(dst_hbm.at[br, w, 2 * g + 1], dstb[1])
            dg0.wait()
            ds0 = pltpu.async_copy(rows[0], acc.at[dstb[0]], ssems[0],
                                   add=True)
            dg1 = pltpu.async_copy(x_hbm.at[br].at[srcb[1]], rows[1], gsem)
            dg1.wait()
            ds1 = pltpu.async_copy(rows[1], acc.at[dstb[1]], ssems[1],
                                   add=True)
            ds0.wait()
            pltpu.sync_copy(src_hbm.at[br, w, 2 * g + 2], srcb[0])
            pltpu.sync_copy(dst_hbm.at[br, w, 2 * g + 2], dstb[0])
            ds1.wait()
            return 0

        lax.fori_loop(0, NSTEPS // 2, pair, 0)
        plsc.subcore_barrier()

        for off, sz in ((0, 128), (128, 128), (256, 128), (384, 128),
                        (512, 120)):
            r0 = s * ZROWS + off
            buf = rows[0].at[pl.ds(0, sz)]
            pltpu.sync_copy(acc.at[pl.ds(r0, sz)], buf)
            pltpu.sync_copy(buf, out_hbm.at[br, c, pl.ds(r0, sz)])


BN = 2000
NBLK = N // BN


def _mm(a, b):
    return jnp.dot(a, b, preferred_element_type=jnp.float32,
                   precision=lax.Precision.HIGHEST)


def _gin_math(x, p0, p1, w1_ref, b1_ref, w2_ref, b2_ref):
    h = x + p0 + p1
    h = _mm(h, w1_ref[...]) + b1_ref[...]
    h = jnp.maximum(h, 0.0)
    h = _mm(h, w2_ref[...]) + b2_ref[...]
    return jnp.maximum(h, 0.0)


def _tc_layer_body(x_ref, p0_ref, p1_ref, w1_ref, b1_ref, w2_ref, b2_ref,
                   o_ref):
    o_ref[0] = _gin_math(x_ref[0], p0_ref[0, 0], p1_ref[0, 0],
                         w1_ref, b1_ref, w2_ref, b2_ref)


_X_SPECS = [
    pl.BlockSpec((1, BN, D), lambda b, i: (b, i, 0)),
    pl.BlockSpec((1, 1, BN, D), lambda b, i: (b, 0, i, 0)),
    pl.BlockSpec((1, 1, BN, D), lambda b, i: (b, 1, i, 0)),
]
_W_SPECS = [
    pl.BlockSpec((D, H), lambda b, i: (0, 0)),
    pl.BlockSpec((1, H), lambda b, i: (0, 0)),
    pl.BlockSpec((H, H), lambda b, i: (0, 0)),
    pl.BlockSpec((1, H), lambda b, i: (0, 0)),
]


def _tc_layer(x3, parts, w1, b1, w2, b2):
    return pl.pallas_call(
        _tc_layer_body,
        grid=(NB, NBLK),
        in_specs=_X_SPECS + _W_SPECS,
        out_specs=pl.BlockSpec((1, BN, H), lambda b, i: (b, i, 0)),
        out_shape=jax.ShapeDtypeStruct((NB, N, H), jnp.float32),
    )(x3, parts, parts, w1, b1, w2, b2)


def _tc_layer_pool_body(x_ref, p0_ref, p1_ref, w1_ref, b1_ref, w2_ref,
                        b2_ref, batch_ref, s_ref, c_ref):
    h = _gin_math(x_ref[0], p0_ref[0, 0], p1_ref[0, 0],
                  w1_ref, b1_ref, w2_ref, b2_ref)

    gids = lax.broadcasted_iota(jnp.int32, (BN, G), 1)
    onehot_t = (gids == batch_ref[0]).astype(jnp.float32)

    @pl.when(pl.program_id(1) == 0)
    def _():
        s_ref[...] = jnp.zeros_like(s_ref)
        c_ref[...] = jnp.zeros_like(c_ref)

    s_ref[0] += lax.dot_general(
        onehot_t, h, (((0,), (0,)), ((), ())),
        preferred_element_type=jnp.float32,
        precision=lax.Precision.HIGHEST)
    c_ref[0] += jnp.sum(onehot_t, axis=0)[None, :]


def _tc_layer_pool(x3, parts, w1, b1, w2, b2, batch3):
    return pl.pallas_call(
        _tc_layer_pool_body,
        grid=(NB, NBLK),
        in_specs=_X_SPECS + _W_SPECS + [
            pl.BlockSpec((1, BN, 1), lambda b, i: (b, i, 0))],
        out_specs=[
            pl.BlockSpec((1, G, H), lambda b, i: (b, 0, 0)),
            pl.BlockSpec((1, 1, G), lambda b, i: (b, 0, 0)),
        ],
        out_shape=[
            jax.ShapeDtypeStruct((NB, G, H), jnp.float32),
            jax.ShapeDtypeStruct((NB, 1, G), jnp.float32),
        ],
    )(x3, parts, parts, w1, b1, w2, b2, batch3)


def _tc_head_body(s3_ref, c3_ref, mw_ref, mb_ref, fw1_ref, fb1_ref,
                  fw2_ref, fb2_ref, o_ref):
    def hg(br):
        cnt = jnp.maximum(c3_ref[br], 1.0)
        pooled = s3_ref[br] / cnt.reshape(G, 1)
        return _mm(pooled, mw_ref[...]) + mb_ref[...]

    acc = (_mm(hg(0), fw1_ref[0:H, :])
           + _mm(hg(1), fw1_ref[H:2 * H, :])
           + _mm(hg(2), fw1_ref[2 * H:3 * H, :]))
    acc = jnp.maximum(acc + fb1_ref[...], 0.0)
    o_ref[...] = _mm(acc, fw2_ref[...]) + fb2_ref[...]


def _tc_head(s3, c3, mlp_W, mlp_b2, fW1, fb1_2, fW2, fb2_2):
    return pl.pallas_call(
        _tc_head_body,
        out_shape=jax.ShapeDtypeStruct((G, NC_OUT), jnp.float32),
    )(s3, c3, mlp_W, mlp_b2, fW1, fb1_2, fW2, fb2_2)


def kernel(x_org, edge_index_org, batch_org, x_c1, edge_index_c1, batch_c1,
           x_c2, edge_index_c2, batch_c2, conv_W1, conv_b1, conv_W2, conv_b2,
           mlp_W, mlp_b, final_W1, final_b1, final_W2, final_b2):
    zeros = jnp.zeros((ZROWS, D), jnp.float32)
    pad_src = jnp.zeros((EPAD - E,), jnp.int32)
    pad_dst = N + (jnp.arange(EPAD - E, dtype=jnp.int32) % (ACC_ROWS - N))

    dummy = jnp.zeros((NW, 1, CHUNK), jnp.int32)

    def prep(ei):
        src = jnp.concatenate([ei[0], pad_src]).reshape(NW, NSTEPS, CHUNK)
        dst = jnp.concatenate([ei[1], pad_dst]).reshape(NW, NSTEPS, CHUNK)
        return (jnp.concatenate([src, dummy], axis=1),
                jnp.concatenate([dst, dummy], axis=1))

    s_o, d_o = prep(edge_index_org)
    s_1, d_1 = prep(edge_index_c1)
    s_2, d_2 = prep(edge_index_c2)
    src3 = jnp.stack([s_o, s_1, s_2])
    dst3 = jnp.stack([d_o, d_1, d_2])
    x3 = jnp.stack([x_org, x_c1, x_c2])
    batch3 = jnp.stack([batch_org, batch_c1, batch_c2]).reshape(NB, N, 1)

    b1r = conv_b1.reshape(3, 1, H)
    b2r = conv_b2.reshape(3, 1, H)

    for l in range(2):
        parts = _sc_agg3(x3, src3, dst3, zeros)
        x3 = _tc_layer(x3, parts, conv_W1[l], b1r[l], conv_W2[l], b2r[l])
    parts = _sc_agg3(x3, src3, dst3, zeros)
    s3, c3 = _tc_layer_pool(x3, parts, conv_W1[2], b1r[2], conv_W2[2],
                            b2r[2], batch3)

    return _tc_head(s3, c3,
                    mlp_W, mlp_b.reshape(1, H),
                    final_W1, final_b1.reshape(1, H),
                    final_W2, final_b2.reshape(1, NC_OUT))


# speedup vs baseline: 1.7857x; 1.3639x over previous
import functools

import jax
import jax.numpy as jnp
from jax import lax
from jax.experimental import pallas as pl
from jax.experimental.pallas import tpu as pltpu
from jax.experimental.pallas import tpu_sc as plsc

N = 10000
E = 320000
D = 128
H = 128
G = 64
NC_OUT = 10

NCORES = 2
NSUB = 16
NW = NCORES * NSUB
CHUNK = 128
NSTEPS = 79
EPW = NSTEPS * CHUNK
EPAD = EPW * NW
ACC_ROWS = 10240
ZROWS = ACC_ROWS // NSUB


@functools.partial(
    pl.kernel,
    out_type=jax.ShapeDtypeStruct((NCORES, ACC_ROWS, D), jnp.float32),
    mesh=plsc.VectorSubcoreMesh(core_axis_name="c", subcore_axis_name="s"),
    scratch_types=[
        pltpu.VMEM((CHUNK,), jnp.int32),
        pltpu.VMEM((CHUNK,), jnp.int32),
        pltpu.VMEM((CHUNK, D), jnp.float32),
        pltpu.VMEM_SHARED((ACC_ROWS, D), jnp.float32),
        pltpu.SemaphoreType.DMA,
    ],
)
def _sc_agg(x_hbm, src_hbm, dst_hbm, zeros_hbm, out_hbm,
            srcbuf, dstbuf, rows, acc, sem):
    c = lax.axis_index("c")
    s = lax.axis_index("s")

    pltpu.sync_copy(zeros_hbm, acc.at[pl.ds(s * ZROWS, ZROWS)])
    plsc.subcore_barrier()

    base = (c * NSUB + s) * EPW

    def step(i, _):
        off = base + i * CHUNK
        pltpu.sync_copy(src_hbm.at[pl.ds(off, CHUNK)], srcbuf)
        pltpu.sync_copy(dst_hbm.at[pl.ds(off, CHUNK)], dstbuf)
        pltpu.async_copy(x_hbm.at[srcbuf], rows, sem).wait()
        pltpu.sync_copy(rows, acc.at[dstbuf], add=True)
        return 0

    lax.fori_loop(0, NSTEPS, step, 0)
    plsc.subcore_barrier()

    for b in range(ZROWS // CHUNK):
        r0 = s * ZROWS + b * CHUNK
        pltpu.sync_copy(acc.at[pl.ds(r0, CHUNK)], rows)
        pltpu.sync_copy(rows, out_hbm.at[c, pl.ds(r0, CHUNK)])


BN = 2000


def _tc_layer_body(x_ref, p0_ref, p1_ref, w1_ref, b1_ref, w2_ref, b2_ref,
                   o_ref):
    h = x_ref[...] + p0_ref[0] + p1_ref[0]
    h = jnp.dot(h, w1_ref[...], preferred_element_type=jnp.float32,
                precision=lax.Precision.HIGHEST) + b1_ref[...]
    h = jnp.maximum(h, 0.0)
    h = jnp.dot(h, w2_ref[...], preferred_element_type=jnp.float32,
                precision=lax.Precision.HIGHEST) + b2_ref[...]
    o_ref[...] = jnp.maximum(h, 0.0)


def _tc_layer(x, parts, w1, b1, w2, b2):
    grid = (N // BN,)
    return pl.pallas_call(
        _tc_layer_body,
        grid=grid,
        in_specs=[
            pl.BlockSpec((BN, D), lambda i: (i, 0)),
            pl.BlockSpec((1, BN, D), lambda i: (0, i, 0)),
            pl.BlockSpec((1, BN, D), lambda i: (1, i, 0)),
            pl.BlockSpec((D, H), lambda i: (0, 0)),
            pl.BlockSpec((1, H), lambda i: (0, 0)),
            pl.BlockSpec((H, H), lambda i: (0, 0)),
            pl.BlockSpec((1, H), lambda i: (0, 0)),
        ],
        out_specs=pl.BlockSpec((BN, H), lambda i: (i, 0)),
        out_shape=jax.ShapeDtypeStruct((N, H), jnp.float32),
    )(x, parts, parts, w1, b1, w2, b2)


def _tc_layer_pool_body(x_ref, p0_ref, p1_ref, w1_ref, b1_ref, w2_ref,
                        b2_ref, batch_ref, s_ref, c_ref):
    h = x_ref[...] + p0_ref[0] + p1_ref[0]
    h = jnp.dot(h, w1_ref[...], preferred_element_type=jnp.float32,
                precision=lax.Precision.HIGHEST) + b1_ref[...]
    h = jnp.maximum(h, 0.0)
    h = jnp.dot(h, w2_ref[...], preferred_element_type=jnp.float32,
                precision=lax.Precision.HIGHEST) + b2_ref[...]
    h = jnp.maximum(h, 0.0)

    gids = lax.broadcasted_iota(jnp.int32, (BN, G), 1)
    onehot_t = (gids == batch_ref[...]).astype(jnp.float32)

    @pl.when(pl.program_id(0) == 0)
    def _():
        s_ref[...] = jnp.zeros_like(s_ref)
        c_ref[...] = jnp.zeros_like(c_ref)

    s_ref[...] += lax.dot_general(
        onehot_t, h, (((0,), (0,)), ((), ())),
        preferred_element_type=jnp.float32,
        precision=lax.Precision.HIGHEST)
    c_ref[...] += jnp.sum(onehot_t, axis=0)[None, :]


def _tc_layer_pool(x, parts, w1, b1, w2, b2, batch2):
    grid = (N // BN,)
    return pl.pallas_call(
        _tc_layer_pool_body,
        grid=grid,
        in_specs=[
            pl.BlockSpec((BN, D), lambda i: (i, 0)),
            pl.BlockSpec((1, BN, D), lambda i: (0, i, 0)),
            pl.BlockSpec((1, BN, D), lambda i: (1, i, 0)),
            pl.BlockSpec((D, H), lambda i: (0, 0)),
            pl.BlockSpec((1, H), lambda i: (0, 0)),
            pl.BlockSpec((H, H), lambda i: (0, 0)),
            pl.BlockSpec((1, H), lambda i: (0, 0)),
            pl.BlockSpec((BN, 1), lambda i: (i, 0)),
        ],
        out_specs=[
            pl.BlockSpec((G, H), lambda i: (0, 0)),
            pl.BlockSpec((1, G), lambda i: (0, 0)),
        ],
        out_shape=[
            jax.ShapeDtypeStruct((G, H), jnp.float32),
            jax.ShapeDtypeStruct((1, G), jnp.float32),
        ],
    )(x, parts, parts, w1, b1, w2, b2, batch2)


def _tc_head_body(s0_ref, c0_ref, s1_ref, c1_ref, s2_ref, c2_ref,
                  mw_ref, mb_ref, fw1_ref, fb1_ref, fw2_ref, fb2_ref,
                  o_ref):
    def hg(s_ref, c_ref):
        cnt = jnp.maximum(c_ref[...], 1.0)
        pooled = s_ref[...] / cnt.reshape(G, 1)
        return jnp.dot(pooled, mw_ref[...],
                       preferred_element_type=jnp.float32,
                       precision=lax.Precision.HIGHEST) + mb_ref[...]

    h0 = hg(s0_ref, c0_ref)
    h1 = hg(s1_ref, c1_ref)
    h2 = hg(s2_ref, c2_ref)
    acc = (jnp.dot(h0, fw1_ref[0:H, :], preferred_element_type=jnp.float32,
                   precision=lax.Precision.HIGHEST)
           + jnp.dot(h1, fw1_ref[H:2 * H, :],
                     preferred_element_type=jnp.float32,
                     precision=lax.Precision.HIGHEST)
           + jnp.dot(h2, fw1_ref[2 * H:3 * H, :],
                     preferred_element_type=jnp.float32,
                     precision=lax.Precision.HIGHEST))
    acc = jnp.maximum(acc + fb1_ref[...], 0.0)
    o_ref[...] = jnp.dot(acc, fw2_ref[...], preferred_element_type=jnp.float32,
                         precision=lax.Precision.HIGHEST) + fb2_ref[...]


def _tc_head(s0, c0, s1, c1, s2, c2, mlp_W, mlp_b2, fW1, fb1_2, fW2, fb2_2):
    return pl.pallas_call(
        _tc_head_body,
        out_shape=jax.ShapeDtypeStruct((G, NC_OUT), jnp.float32),
    )(s0, c0, s1, c1, s2, c2, mlp_W, mlp_b2, fW1, fb1_2, fW2, fb2_2)


def kernel(x_org, edge_index_org, batch_org, x_c1, edge_index_c1, batch_c1,
           x_c2, edge_index_c2, batch_c2, conv_W1, conv_b1, conv_W2, conv_b2,
           mlp_W, mlp_b, final_W1, final_b1, final_W2, final_b2):
    zeros = jnp.zeros((ZROWS, D), jnp.float32)
    pad_src = jnp.zeros((EPAD - E,), jnp.int32)
    pad_dst = jnp.full((EPAD - E,), N, jnp.int32)

    b1r = conv_b1.reshape(3, 1, H)
    b2r = conv_b2.reshape(3, 1, H)

    def branch(x, ei, batch):
        src = jnp.concatenate([ei[0], pad_src])
        dst = jnp.concatenate([ei[1], pad_dst])
        batch2 = batch.reshape(N, 1)
        for l in range(2):
            parts = _sc_agg(x, src, dst, zeros)
            x = _tc_layer(x, parts, conv_W1[l], b1r[l], conv_W2[l], b2r[l])
        parts = _sc_agg(x, src, dst, zeros)
        return _tc_layer_pool(x, parts, conv_W1[2], b1r[2], conv_W2[2],
                              b2r[2], batch2)

    s0, c0 = branch(x_org, edge_index_org, batch_org)
    s1, c1 = branch(x_c1, edge_index_c1, batch_c1)
    s2, c2 = branch(x_c2, edge_index_c2, batch_c2)

    return _tc_head(s0, c0, s1, c1, s2, c2,
                    mlp_W, mlp_b.reshape(1, H),
                    final_W1, final_b1.reshape(1, H),
                    final_W2, final_b2.reshape(1, NC_OUT))
